# Initial kernel scaffold; baseline (speedup 1.0000x reference)
import jax
import jax.numpy as jnp
from jax.experimental import pallas as pl


def kernel(base_x, base_edge_index, base_edge_weight, local_x, local_adj_index, local_adj_value, copy2orig, W_base, b_base, W_local, b_local, gn_base_weight, gn_base_bias, gn_base_ms, gn_local_weight, gn_local_bias, gn_local_ms):
    raise NotImplementedError("write your pallas kernel here")


# speedup vs baseline: 6.8518x; 6.8518x over previous
import dataclasses
import functools

import jax
import jax.numpy as jnp
from jax import lax
from jax.experimental import pallas as pl
from jax.experimental.pallas import tpu as pltpu
from jax.experimental.pallas import tpu_sc as plsc

NC, NS = 2, 16

_CP = pltpu.CompilerParams()
_fields = pltpu.CompilerParams.__dataclass_fields__
if "needs_layout_passes" in _fields:
    _CP = dataclasses.replace(_CP, needs_layout_passes=False)
if "use_tc_tiling_on_sc" in _fields:
    _CP = dataclasses.replace(_CP, use_tc_tiling_on_sc=False)

_MESH = dict(
    mesh=plsc.VectorSubcoreMesh(core_axis_name="c", subcore_axis_name="s"),
    compiler_params=_CP,
)

_F32 = jnp.float32
_I32 = jnp.int32


def _tc_linear2(x, W, b):
    n = x.shape[0]

    def body(x_ref, w_ref, b_ref, o_ref):
        y = lax.dot_general(x_ref[...], w_ref[...], (((1,), (1,)), ((), ())),
                            preferred_element_type=_F32)
        y = y + b_ref[...][None, :]
        o_ref[0] = y[:, :64]
        o_ref[1] = y[:, 64:]

    return pl.pallas_call(
        body, out_shape=jax.ShapeDtypeStruct((2, n, 64), _F32))(x, W, b)


def _tc_dinv(degp):
    def body(d_ref, o_ref):
        d = jnp.sum(d_ref[...], axis=0)
        d = jnp.where(d < 0.5, d + 1.0, d)
        o_ref[...] = lax.rsqrt(d)

    return pl.pallas_call(
        body, out_shape=jax.ShapeDtypeStruct((80, 128), _F32))(degp)


def _tc_graphnorm_relu(parts, w2, b2, ms2, n_real):
    R = sum(p.shape[1] for p in parts)
    inv_n = 1.0 / float(n_real)

    def body(*refs):
        in_refs, o_ref = refs[:-4], refs[-1]
        w_ref, b_ref, m_ref = refs[-4], refs[-3], refs[-2]
        for h in range(2):
            if len(in_refs) == 1:
                x = in_refs[0][h]
            else:
                x = jnp.concatenate([r[h] for r in in_refs], axis=0)
            sx = jnp.sum(x, axis=0) * inv_n
            sxx = jnp.sum(x * x, axis=0) * inv_n
            c = sx * m_ref[h]
            var = sxx - 2.0 * c * sx + c * c
            scale = lax.rsqrt(var + 1e-5) * w_ref[h]
            y = (x - c[None, :]) * scale[None, :] + b_ref[h][None, :]
            o_ref[h] = jnp.maximum(y, 0.0)

    return pl.pallas_call(
        body, out_shape=jax.ShapeDtypeStruct((2, R, 64), _F32))(
            *parts, w2, b2, ms2)


def _tc_final_base(bh2, s2, cnt2):
    B = 2000

    def body(bh_ref, s_ref, c_ref, o_ref):
        cnt = c_ref[0] + c_ref[1]
        inv = 0.2 / jnp.maximum(cnt[:, 0:1], 1.0)
        o_ref[...] = jnp.concatenate(
            [0.8 * bh_ref[h] + s_ref[h] * inv for h in range(2)], axis=1)

    return pl.pallas_call(
        body,
        grid=(10000 // B,),
        in_specs=[pl.BlockSpec((2, B, 64), lambda i: (0, i, 0))] * 3,
        out_specs=pl.BlockSpec((B, 128), lambda i: (i, 0)),
        out_shape=jax.ShapeDtypeStruct((10000, 128), _F32))(bh2, s2, cnt2)


def _tc_assemble(lm2):
    B = 2000

    def body(l_ref, o_ref):
        o_ref[...] = jnp.concatenate([l_ref[0], l_ref[1]], axis=1)

    return pl.pallas_call(
        body,
        grid=(20000 // B,),
        in_specs=[pl.BlockSpec((2, B, 64), lambda i: (0, i, 0))],
        out_specs=pl.BlockSpec((B, 128), lambda i: (i, 0)),
        out_shape=jax.ShapeDtypeStruct((20000, 128), _F32))(lm2)


def _sc_deg_cnt(row_r, ew_r, d3):
    @functools.partial(
        pl.kernel,
        out_type=(jax.ShapeDtypeStruct((32, 640, 16), _F32),
                  jax.ShapeDtypeStruct((NC, 10048, 64), _F32)),
        scratch_types=[
            pltpu.VMEM((625, 16), _I32),
            pltpu.VMEM((625, 16), _F32),
            pltpu.VMEM((8, 80), _I32),
            pltpu.VMEM((640, 16), _F32),
            pltpu.VMEM((80, 64), _F32),
            pltpu.VMEM((157, 64), _F32),
            pltpu.VMEM_SHARED((10048, 64), _F32),
            pltpu.SemaphoreType.DMA,
        ],
        **_MESH,
    )
    def k(row_hbm, ew_hbm, d_hbm, degp_hbm, cnt_hbm, idx_v, val_v, didx,
          part_v, ones_v, zbuf, cnt_sh, sem):
        c = lax.axis_index("c")
        s = lax.axis_index("s")
        t = c * NS + s
        zero = jnp.zeros((16,), _F32)
        one = jnp.full((16,), 1.0, _F32)

        @pl.loop(0, 640)
        def _(i):
            part_v[i] = zero

        @pl.loop(0, 157)
        def _(i):
            for j in range(4):
                zbuf[i, pl.ds(j * 16, 16)] = zero

        @pl.loop(0, 80)
        def _(i):
            for j in range(4):
                ones_v[i, pl.ds(j * 16, 16)] = one

        for z in range(4):
            pltpu.sync_copy(zbuf, cnt_sh.at[pl.ds(s * 628 + z * 157, 157)])
        pltpu.sync_copy(row_hbm.at[t], idx_v)
        pltpu.sync_copy(ew_hbm.at[t], val_v)
        pltpu.sync_copy(d_hbm.at[t], didx)
        plsc.subcore_barrier()

        @pl.loop(0, 625)
        def _(i):
            r = idx_v[i]
            v = val_v[i]
            rhi = lax.shift_right_logical(r, 4)
            rlo = lax.bitwise_and(r, 15)
            plsc.addupdate_scatter(part_v, [rhi, rlo], v)

        pltpu.sync_copy(part_v, degp_hbm.at[t])

        @pl.loop(0, 8)
        def _(ci):
            pltpu.sync_copy(ones_v, cnt_sh.at[didx.at[ci]], add=True)

        plsc.subcore_barrier()
        pltpu.sync_copy(cnt_sh.at[pl.ds(s * 628, 628)],
                        cnt_hbm.at[c].at[pl.ds(s * 628, 628)])

    return k(row_r, ew_r, d3)


def _sc_edge_agg(tab2, dst3, src3, val3, dinv, acc_rows, out_rows, cpt,
                 dst_off):
    K = 80
    zr = {625: 125, 640: 128}[out_rows // 16]
    stripe = out_rows // 16
    scratch = [
        pltpu.VMEM((cpt, K), _I32),
        pltpu.VMEM((cpt, K), _I32),
        pltpu.VMEM((cpt, K), _F32),
        pltpu.VMEM((K + 16,), _F32),
        pltpu.VMEM((K, 64), _F32),
        pltpu.VMEM((zr, 64), _F32),
        pltpu.VMEM_SHARED((acc_rows, 64), _F32),
        pltpu.SemaphoreType.DMA,
    ]
    if dinv is not None:
        scratch.insert(3, pltpu.VMEM((dinv.shape[0],), _F32))

    @functools.partial(
        pl.kernel,
        out_type=jax.ShapeDtypeStruct((NC, out_rows, 64), _F32),
        scratch_types=scratch,
        **_MESH,
    )
    def k(tab_hbm, dst_hbm, src_hbm, val_hbm, *rest):
        if dinv is not None:
            (dinv_hbm, out_hbm, dstv, srcv, valv, dinvv, wbuf, gbuf, zbuf,
             acc_sh, sem) = rest
        else:
            (out_hbm, dstv, srcv, valv, wbuf, gbuf, zbuf, acc_sh, sem) = rest
        c = lax.axis_index("c")
        s = lax.axis_index("s")
        zero = jnp.zeros((16,), _F32)

        @pl.loop(0, zr)
        def _(i):
            for j in range(4):
                zbuf[i, pl.ds(j * 16, 16)] = zero

        for z in range(stripe // zr):
            pltpu.sync_copy(zbuf, acc_sh.at[pl.ds(s * stripe + z * zr, zr)])

        pltpu.sync_copy(dst_hbm.at[s], dstv)
        pltpu.sync_copy(src_hbm.at[s], srcv)
        pltpu.sync_copy(val_hbm.at[s], valv)
        if dinv is not None:
            pltpu.sync_copy(dinv_hbm, dinvv)
        plsc.subcore_barrier()

        @pl.loop(0, cpt)
        def _(ci):
            pltpu.sync_copy(tab_hbm.at[c].at[srcv.at[ci]], gbuf)
            for j in range(K // 16):
                sl = pl.ds(j * 16, 16)
                ww = valv[ci, sl]
                if dinv is not None:
                    ww = (plsc.load_gather(dinvv, [dstv[ci, sl]]) * ww
                          * plsc.load_gather(dinvv, [srcv[ci, sl]]))
                wbuf[pl.ds(16 + j * 16, 16)] = ww
                if dst_off is not None:
                    d = dstv[ci, sl] - dst_off
                    ok = jnp.logical_and(d >= 0, d < out_rows)
                    dstv[ci, sl] = jnp.where(ok, d, out_rows)
            for i in range(K):
                wv = plsc.load_gather(wbuf, [jnp.full((16,), 16 + i, _I32)])
                for j in range(4):
                    sl = pl.ds(j * 16, 16)
                    gbuf[i, sl] = gbuf[i, sl] * wv
            pltpu.sync_copy(gbuf, acc_sh.at[dstv.at[ci]], add=True)

        plsc.subcore_barrier()
        pltpu.sync_copy(acc_sh.at[pl.ds(s * stripe, stripe)],
                        out_hbm.at[c].at[pl.ds(s * stripe, stripe)])

    args = (tab2, dst3, src3, val3) + ((dinv,) if dinv is not None else ())
    return k(*args)


def _sc_mix(bh2, lh2, g3, d3):
    K, CH = 80, 16

    @functools.partial(
        pl.kernel,
        out_type=(jax.ShapeDtypeStruct((NC, 20480, 64), _F32),
                  jax.ShapeDtypeStruct((NC, 10048, 64), _F32)),
        scratch_types=[
            pltpu.VMEM((CH, K), _I32),
            pltpu.VMEM((CH, K), _I32),
            pltpu.VMEM((K, 64), _F32),
            pltpu.VMEM((K, 64), _F32),
            pltpu.VMEM((157, 64), _F32),
            pltpu.VMEM_SHARED((10048, 64), _F32),
            pltpu.SemaphoreType.DMA,
        ],
        **_MESH,
    )
    def k(bh_hbm, lh_hbm, g_hbm, d_hbm, lm_hbm, s_hbm,
          gidx, didx, gbuf, lbuf, zbuf, s_sh, sem):
        c = lax.axis_index("c")
        s = lax.axis_index("s")
        base = s * (CH * K)
        zero = jnp.zeros((16,), _F32)

        @pl.loop(0, 157)
        def _(i):
            for j in range(4):
                zbuf[i, pl.ds(j * 16, 16)] = zero

        for z in range(4):
            pltpu.sync_copy(zbuf, s_sh.at[pl.ds(s * 628 + z * 157, 157)])
        pltpu.sync_copy(g_hbm.at[s], gidx)
        pltpu.sync_copy(d_hbm.at[s], didx)
        plsc.subcore_barrier()

        @pl.loop(0, CH)
        def _(ci):
            pltpu.sync_copy(bh_hbm.at[c].at[gidx.at[ci]], gbuf)
            pltpu.sync_copy(lh_hbm.at[c].at[pl.ds(base + ci * K, K)], lbuf)
            for i in range(K):
                for j in range(4):
                    sl = pl.ds(j * 16, 16)
                    lbuf[i, sl] = 0.8 * lbuf[i, sl] + 0.2 * gbuf[i, sl]
            pltpu.sync_copy(lbuf, lm_hbm.at[c].at[pl.ds(base + ci * K, K)])
            pltpu.sync_copy(lbuf, s_sh.at[didx.at[ci]], add=True)

        plsc.subcore_barrier()
        pltpu.sync_copy(s_sh.at[pl.ds(s * 628, 628)],
                        s_hbm.at[c].at[pl.ds(s * 628, 628)])

    return k(bh2, lh2, g3, d3)


def kernel(base_x, base_edge_index, base_edge_weight, local_x,
           local_adj_index, local_adj_value, copy2orig, W_base, b_base,
           W_local, b_local, gn_base_weight, gn_base_bias, gn_base_ms,
           gn_local_weight, gn_local_bias, gn_local_ms):
    N = base_x.shape[0]
    M = local_x.shape[0]
    MP = 20480

    row = base_edge_index[0].astype(_I32)
    col = base_edge_index[1].astype(_I32)
    rL = local_adj_index[0].astype(_I32)
    cL = local_adj_index[1].astype(_I32)
    c2o = copy2orig.astype(_I32)
    pad = MP - M
    c2o_g = jnp.concatenate([c2o, jnp.zeros((pad,), _I32)])
    c2o_d = jnp.concatenate([c2o, 10000 + (jnp.arange(pad, dtype=_I32) % 48)])

    bx2 = _tc_linear2(base_x, W_base, b_base)
    lx2 = _tc_linear2(jnp.pad(local_x, ((0, pad), (0, 0))),
                      W_local, b_local)

    degp, cnt2 = _sc_deg_cnt(row.reshape(32, 625, 16),
                             base_edge_weight.reshape(32, 625, 16),
                             c2o_d.reshape(32, 8, 80))
    dinv = _tc_dinv(degp.reshape(32, 80, 128)).reshape(-1)

    acc2 = _sc_edge_agg(bx2, row.reshape(16, 250, 80),
                        col.reshape(16, 250, 80),
                        base_edge_weight.reshape(16, 250, 80), dinv,
                        acc_rows=10000, out_rows=10000, cpt=250, dst_off=None)
    bh2 = _tc_graphnorm_relu([acc2], gn_base_weight.reshape(2, 64),
                             gn_base_bias.reshape(2, 64),
                             gn_base_ms.reshape(2, 64), N)

    accL = [
        _sc_edge_agg(lx2, rL.reshape(16, 125, 80), cL.reshape(16, 125, 80),
                     local_adj_value.reshape(16, 125, 80), None,
                     acc_rows=10248, out_rows=10240, cpt=125, dst_off=off)
        for off in (0, 10240)
    ]
    lh2 = _tc_graphnorm_relu(accL, gn_local_weight.reshape(2, 64),
                             gn_local_bias.reshape(2, 64),
                             gn_local_ms.reshape(2, 64), M)

    lm2, s2 = _sc_mix(bh2, lh2, c2o_g.reshape(16, 16, 80),
                      c2o_d.reshape(16, 16, 80))
    base_mixed = _tc_final_base(bh2, s2, cnt2)
    local_mixed = _tc_assemble(lm2)
    return (base_mixed, local_mixed)
